# baseline (speedup 1.0000x reference)
import functools

import jax
import jax.numpy as jnp
from jax import lax
from jax.experimental import pallas as pl
from jax.experimental.pallas import tpu as pltpu
from jax.experimental.pallas import tpu_sc as plsc

SEQ = 8192
EMB = 768
L = 16
NVEC = EMB // L
NC = 2
NS = 16
NW = NC * NS
TOK_PER_W = SEQ // NW
CH = 64
NCH = TOK_PER_W // CH
EPS = 1e-5


def _lane_sum(v):
    lanes = lax.iota(jnp.int32, L)
    for k in (8, 4, 2, 1):
        v = v + v.at[lanes ^ k].get(mode="promise_in_bounds")
    return v


def _rsqrt(v):
    i = lax.bitcast_convert_type(v, jnp.int32)
    i = 0x5F3759DF - lax.shift_right_arithmetic(i, 1)
    y = lax.bitcast_convert_type(i, jnp.float32)
    for _ in range(3):
        y = y * (1.5 - 0.5 * v * y * y)
    return y


_mesh = plsc.VectorSubcoreMesh(core_axis_name="c", subcore_axis_name="s")


@functools.partial(
    pl.kernel,
    mesh=_mesh,
    out_type=jax.ShapeDtypeStruct((SEQ, EMB), jnp.float32),
    scratch_types=[
        pltpu.VMEM((TOK_PER_W,), jnp.int32),
        pltpu.VMEM((CH, EMB), jnp.float32),
        pltpu.VMEM((CH, EMB), jnp.float32),
        pltpu.VMEM((EMB,), jnp.float32),
        pltpu.VMEM((EMB,), jnp.float32),
        pltpu.SemaphoreType.DMA,
    ],
)
def _encode(ids_hbm, tab_hbm, pos_hbm, gam_hbm, bet_hbm, out_hbm,
            idx_v, x_v, p_v, gam_v, bet_v, sem):
    wid = lax.axis_index("s") * NC + lax.axis_index("c")
    base = wid * TOK_PER_W
    pltpu.sync_copy(ids_hbm.at[pl.ds(base, TOK_PER_W)], idx_v)
    pltpu.sync_copy(gam_hbm, gam_v)
    pltpu.sync_copy(bet_hbm, bet_v)

    for c in range(NCH):
        cbase = base + c * CH
        pltpu.async_copy(tab_hbm.at[idx_v.at[pl.ds(c * CH, CH)]], x_v, sem).wait()
        pltpu.sync_copy(pos_hbm.at[pl.ds(cbase, CH)], p_v)

        def _row(r, carry):
            s = jnp.zeros((L,), jnp.float32)
            ss = jnp.zeros((L,), jnp.float32)
            for j in range(NVEC):
                x = x_v[r, pl.ds(j * L, L)] + p_v[r, pl.ds(j * L, L)]
                x_v[r, pl.ds(j * L, L)] = x
                s = s + x
                ss = ss + x * x
            mean = _lane_sum(s) * (1.0 / EMB)
            ex2 = _lane_sum(ss) * (1.0 / EMB)
            inv = _rsqrt(ex2 - mean * mean + EPS)
            for j in range(NVEC):
                sl = pl.ds(j * L, L)
                x_v[r, sl] = (x_v[r, sl] - mean) * inv * gam_v[sl] + bet_v[sl]
            return carry

        lax.fori_loop(0, CH, _row, 0)
        pltpu.sync_copy(x_v, out_hbm.at[pl.ds(cbase, CH)])


def kernel(token_ids, token_table, pos_table, ln_gamma, ln_beta):
    out = _encode(token_ids.astype(jnp.int32), token_table, pos_table,
                  ln_gamma, ln_beta)
    return out[None]


# speedup vs baseline: 1.1270x; 1.1270x over previous
import functools

import jax
import jax.numpy as jnp
from jax import lax
from jax.experimental import pallas as pl
from jax.experimental.pallas import tpu as pltpu
from jax.experimental.pallas import tpu_sc as plsc

SEQ = 8192
EMB = 768
L = 16
NVEC = EMB // L
NC = 2
NS = 16
NW = NC * NS
TOK_PER_W = SEQ // NW
CH = 32
NCH = TOK_PER_W // CH
EPS = 1e-5


def _lane_sum(v):
    lanes = lax.iota(jnp.int32, L)
    for k in (8, 4, 2, 1):
        v = v + v.at[lanes ^ k].get(mode="promise_in_bounds")
    return v


def _rsqrt(v):
    i = lax.bitcast_convert_type(v, jnp.int32)
    i = 0x5F3759DF - lax.shift_right_arithmetic(i, 1)
    y = lax.bitcast_convert_type(i, jnp.float32)
    for _ in range(3):
        y = y * (1.5 - 0.5 * v * y * y)
    return y


_mesh = plsc.VectorSubcoreMesh(core_axis_name="c", subcore_axis_name="s")


@functools.partial(
    pl.kernel,
    mesh=_mesh,
    out_type=jax.ShapeDtypeStruct((SEQ, EMB), jnp.float32),
    scratch_types=[
        pltpu.VMEM((TOK_PER_W,), jnp.int32),
        pltpu.VMEM((CH, EMB), jnp.float32),
        pltpu.VMEM((CH, EMB), jnp.float32),
        pltpu.VMEM((CH, EMB), jnp.float32),
        pltpu.VMEM((CH, EMB), jnp.float32),
        pltpu.VMEM((CH, EMB), jnp.float32),
        pltpu.VMEM((EMB,), jnp.float32),
        pltpu.VMEM((EMB,), jnp.float32),
        pltpu.SemaphoreType.DMA,
        pltpu.SemaphoreType.DMA,
        pltpu.SemaphoreType.DMA,
        pltpu.SemaphoreType.DMA,
        pltpu.SemaphoreType.DMA,
        pltpu.SemaphoreType.DMA,
        pltpu.SemaphoreType.DMA,
        pltpu.SemaphoreType.DMA,
    ],
)
def _encode(ids_hbm, tab_hbm, pos_hbm, gam_hbm, bet_hbm, out_hbm,
            idx_v, tok0, tok1, tok2, pos0, pos1, gam_v, bet_v,
            ts0, ts1, ts2, ps0, ps1, os0, os1, os2):
    wid = lax.axis_index("s") * NC + lax.axis_index("c")
    base = wid * TOK_PER_W
    tok = (tok0, tok1, tok2)
    posb = (pos0, pos1)
    tsem = (ts0, ts1, ts2)
    psem = (ps0, ps1)
    osem = (os0, os1, os2)

    pltpu.sync_copy(ids_hbm.at[pl.ds(base, TOK_PER_W)], idx_v)
    pltpu.sync_copy(gam_hbm, gam_v)
    pltpu.sync_copy(bet_hbm, bet_v)

    def start_chunk(c):
        b, p = c % 3, c % 2
        ht = pltpu.async_copy(
            tab_hbm.at[idx_v.at[pl.ds(c * CH, CH)]], tok[b], tsem[b])
        hp = pltpu.async_copy(
            pos_hbm.at[pl.ds(base + c * CH, CH)], posb[p], psem[p])
        return ht, hp

    h_tok = [None] * 3
    h_pos = [None] * 2
    h_out = [None] * 3
    h_tok[0], h_pos[0] = start_chunk(0)

    for c in range(NCH):
        b, p = c % 3, c % 2
        if c + 1 < NCH:
            nb = (c + 1) % 3
            if h_out[nb] is not None:
                h_out[nb].wait()
                h_out[nb] = None
            h_tok[nb], h_pos[(c + 1) % 2] = start_chunk(c + 1)
        h_tok[b].wait()
        h_pos[p].wait()

        x_v, p_v = tok[b], posb[p]

        def _row(r, carry):
            acc = [jnp.zeros((L,), jnp.float32) for _ in range(8)]
            for j in range(NVEC):
                sl = pl.ds(j * L, L)
                x = x_v[r, sl] + p_v[r, sl]
                x_v[r, sl] = x
                k = j & 3
                acc[k] = acc[k] + x
                acc[4 + k] = acc[4 + k] + x * x
            s = (acc[0] + acc[1]) + (acc[2] + acc[3])
            ss = (acc[4] + acc[5]) + (acc[6] + acc[7])
            mean = _lane_sum(s) * (1.0 / EMB)
            ex2 = _lane_sum(ss) * (1.0 / EMB)
            inv = _rsqrt(ex2 - mean * mean + EPS)
            c2 = -mean * inv
            for j in range(NVEC):
                sl = pl.ds(j * L, L)
                x_v[r, sl] = (x_v[r, sl] * inv + c2) * gam_v[sl] + bet_v[sl]
            return carry

        lax.fori_loop(0, CH, _row, 0)
        h_out[b] = pltpu.async_copy(
            x_v, out_hbm.at[pl.ds(base + c * CH, CH)], osem[b])

    for h in h_out:
        if h is not None:
            h.wait()


def kernel(token_ids, token_table, pos_table, ln_gamma, ln_beta):
    out = _encode(token_ids.astype(jnp.int32), token_table, pos_table,
                  ln_gamma, ln_beta)
    return out[None]


# speedup vs baseline: 1.1899x; 1.0558x over previous
import functools

import jax
import jax.numpy as jnp
from jax import lax
from jax.experimental import pallas as pl
from jax.experimental.pallas import tpu as pltpu
from jax.experimental.pallas import tpu_sc as plsc

SEQ = 8192
EMB = 768
L = 16
NVEC = EMB // L
NC = 2
NS = 16
NW = NC * NS
TOK_PER_W = SEQ // NW
CH = 32
NCH = TOK_PER_W // CH
JG = 12
NJG = NVEC // JG
NG = 12
NNG = NVEC // NG
EPS = 1e-5


def _lane_sum(v):
    lanes = lax.iota(jnp.int32, L)
    for k in (8, 4, 2, 1):
        v = v + v.at[lanes ^ k].get(mode="promise_in_bounds")
    return v


def _rsqrt(v):
    i = lax.bitcast_convert_type(v, jnp.int32)
    i = 0x5F3759DF - lax.shift_right_arithmetic(i, 1)
    y = lax.bitcast_convert_type(i, jnp.float32)
    for _ in range(3):
        y = y * (1.5 - 0.5 * v * y * y)
    return y


_mesh = plsc.VectorSubcoreMesh(core_axis_name="c", subcore_axis_name="s")


@functools.partial(
    pl.kernel,
    mesh=_mesh,
    out_type=jax.ShapeDtypeStruct((SEQ, EMB), jnp.float32),
    scratch_types=[
        pltpu.VMEM((TOK_PER_W,), jnp.int32),
        pltpu.VMEM((CH, EMB), jnp.float32),
        pltpu.VMEM((CH, EMB), jnp.float32),
        pltpu.VMEM((CH, EMB), jnp.float32),
        pltpu.VMEM((EMB,), jnp.float32),
        pltpu.VMEM((EMB,), jnp.float32),
        pltpu.VMEM((CH, L), jnp.float32),
        pltpu.VMEM((CH, L), jnp.float32),
        pltpu.SemaphoreType.DMA,
        pltpu.SemaphoreType.DMA,
        pltpu.SemaphoreType.DMA,
        pltpu.SemaphoreType.DMA,
        pltpu.SemaphoreType.DMA,
    ],
)
def _encode(ids_hbm, tab_hbm, pos_hbm, gam_hbm, bet_hbm, out_hbm,
            idx_v, tok0, tok1, pos_v, gam_v, bet_v, sc_v, of_v,
            ts0, ts1, ps0, os0, os1):
    wid = lax.axis_index("s") * NC + lax.axis_index("c")
    base = wid * TOK_PER_W
    tok = (tok0, tok1)
    tsem = (ts0, ts1)
    osem = (os0, os1)

    pltpu.sync_copy(ids_hbm.at[pl.ds(base, TOK_PER_W)], idx_v)
    pltpu.sync_copy(gam_hbm, gam_v)
    pltpu.sync_copy(bet_hbm, bet_v)

    def start_tok(c):
        return pltpu.async_copy(
            tab_hbm.at[idx_v.at[pl.ds(c * CH, CH)]], tok[c % 2], tsem[c % 2])

    def start_pos(c):
        return pltpu.async_copy(
            pos_hbm.at[pl.ds(base + c * CH, CH)], pos_v, ps0)

    h_tok = [None, None]
    h_out = [None, None]
    h_tok[0] = start_tok(0)
    h_pos = start_pos(0)

    for c in range(NCH):
        b = c % 2
        if c + 1 < NCH:
            nb = 1 - b
            if h_out[nb] is not None:
                h_out[nb].wait()
                h_out[nb] = None
            h_tok[nb] = start_tok(c + 1)
        h_tok[b].wait()
        h_pos.wait()

        x_v = tok[b]

        @plsc.parallel_loop(0, CH)
        def _stats(r):
            def _grp(g, acc):
                s, ss = acc
                gbase = g * (JG * L)
                for jj in range(JG):
                    sl = pl.ds(gbase + jj * L, L)
                    x = x_v[r, sl] + pos_v[r, sl]
                    x_v[r, sl] = x
                    s = s + x
                    ss = ss + x * x
                return s, ss

            z = jnp.zeros((L,), jnp.float32)
            s, ss = lax.fori_loop(0, NJG, _grp, (z, z))
            mean = _lane_sum(s) * (1.0 / EMB)
            ex2 = _lane_sum(ss) * (1.0 / EMB)
            inv = _rsqrt(ex2 - mean * mean + EPS)
            sc_v[r] = inv
            of_v[r] = -mean * inv

        if c + 1 < NCH:
            h_pos = start_pos(c + 1)

        for g in range(NNG):
            gbase = g * (NG * L)
            gams = [gam_v[pl.ds(gbase + jj * L, L)] for jj in range(NG)]
            bets = [bet_v[pl.ds(gbase + jj * L, L)] for jj in range(NG)]

            @plsc.parallel_loop(0, CH)
            def _norm(r):
                inv = sc_v[r]
                c2 = of_v[r]
                for jj in range(NG):
                    sl = pl.ds(gbase + jj * L, L)
                    x_v[r, sl] = (x_v[r, sl] * inv + c2) * gams[jj] + bets[jj]

        h_out[b] = pltpu.async_copy(
            x_v, out_hbm.at[pl.ds(base + c * CH, CH)], osem[b])

    for h in h_out:
        if h is not None:
            h.wait()


def kernel(token_ids, token_table, pos_table, ln_gamma, ln_beta):
    out = _encode(token_ids.astype(jnp.int32), token_table, pos_table,
                  ln_gamma, ln_beta)
    return out[None]
